# baseline (speedup 1.0000x reference)
import jax
import jax.numpy as jnp
from jax import lax
from jax.experimental import pallas as pl
from jax.experimental.pallas import tpu as pltpu

_TOKENS = 1024
_D = 2048
_V = 32768
_BN = 1024
_NT = _V // _BN
_NM = 2
_BM = _TOKENS // _NM
_BN2 = 2048
_NT2 = _V // _BN2

_KS0 = 0
_KS1 = 42
_KS2 = _KS0 ^ _KS1 ^ 0x1BD11BDA


def _threefry_bits(flat):
    ks = (jnp.uint32(_KS0), jnp.uint32(_KS1), jnp.uint32(_KS2))
    rots = ((13, 15, 26, 6), (17, 29, 16, 24))
    x0 = jnp.full_like(flat, ks[0])
    x1 = flat + ks[1]
    for i in range(5):
        for r in rots[i % 2]:
            x0 = x0 + x1
            x1 = lax.shift_left(x1, jnp.uint32(r)) | lax.shift_right_logical(
                x1, jnp.uint32(32 - r))
            x1 = x0 ^ x1
        x0 = x0 + ks[(i + 1) % 3]
        x1 = x1 + ks[(i + 2) % 3] + jnp.uint32(i + 1)
    return x0 ^ x1


def _gumbel(i, j):
    r = lax.broadcasted_iota(jnp.uint32, (_BM, _BN), 0)
    c = lax.broadcasted_iota(jnp.uint32, (_BM, _BN), 1)
    flat = (r + jnp.asarray(i * _BM, jnp.uint32)) * jnp.uint32(_V) \
        + c + jnp.asarray(j * _BN, jnp.uint32)
    bits = _threefry_bits(flat)
    fb = lax.shift_right_logical(bits, jnp.uint32(9)) | jnp.uint32(0x3F800000)
    u = lax.bitcast_convert_type(fb, jnp.float32) - jnp.float32(1.0)
    mn = jnp.float32(1e-10)
    u = jnp.maximum(mn, u * (jnp.float32(1.0) - mn) + mn)
    return -jnp.log(-jnp.log(u))


def _main_kernel(x_ref, w_ref, b_ref, logits_ref, ent_ref, idx_ref,
                 m_ref, s_ref, t_ref, bv_ref, bi_ref):
    i = pl.program_id(0)
    j = pl.program_id(1)
    l = jnp.dot(x_ref[...], w_ref[...],
                preferred_element_type=jnp.float32) + b_ref[...]
    logits_ref[...] = l

    z = l + _gumbel(i, j)
    ztm = jnp.max(z, axis=1, keepdims=True)
    cg = lax.broadcasted_iota(jnp.int32, (_BM, _BN), 1) + j * _BN
    zarg = jnp.min(jnp.where(z >= ztm, cg, jnp.int32(2**31 - 1)),
                   axis=1, keepdims=True)

    @pl.when(j == 0)
    def _():
        m_ref[...] = jnp.full_like(m_ref, -jnp.inf)
        s_ref[...] = jnp.zeros_like(s_ref)
        t_ref[...] = jnp.zeros_like(t_ref)
        bv_ref[...] = jnp.full_like(bv_ref, -jnp.inf)
        bi_ref[...] = jnp.zeros_like(bi_ref)

    m_old = m_ref[...]
    m_new = jnp.maximum(m_old, jnp.max(l, axis=1, keepdims=True))
    e = jnp.exp(l - m_new)
    alpha = jnp.exp(m_old - m_new)
    s_ref[...] = s_ref[...] * alpha + jnp.sum(e, axis=1, keepdims=True)
    t_ref[...] = t_ref[...] * alpha + jnp.sum(l * e, axis=1, keepdims=True)
    m_ref[...] = m_new

    better = ztm > bv_ref[...]
    bv_ref[...] = jnp.where(better, ztm, bv_ref[...])
    bi_ref[...] = jnp.where(better, zarg, bi_ref[...])

    @pl.when(j == _NT - 1)
    def _():
        s = s_ref[...]
        ent_ref[...] = m_ref[...] + jnp.log(s) - t_ref[...] / s
        idx_ref[...] = bi_ref[...]


def _onehot_kernel(idx_ref, out_ref):
    j = pl.program_id(0)
    cg = lax.broadcasted_iota(jnp.int32, (_TOKENS, _BN2), 1) + j * _BN2
    out_ref[...] = (cg == idx_ref[...]).astype(jnp.float32)


def kernel(x, W, b):
    b2d = b.reshape(1, _V)
    logits, ent, idx = pl.pallas_call(
        _main_kernel,
        grid=(_NM, _NT),
        in_specs=[
            pl.BlockSpec((_BM, _D), lambda i, j: (i, 0)),
            pl.BlockSpec((_D, _BN), lambda i, j: (0, j)),
            pl.BlockSpec((1, _BN), lambda i, j: (0, j)),
        ],
        out_specs=[
            pl.BlockSpec((_BM, _BN), lambda i, j: (i, j)),
            pl.BlockSpec((_BM, 1), lambda i, j: (i, 0)),
            pl.BlockSpec((_BM, 1), lambda i, j: (i, 0)),
        ],
        out_shape=[
            jax.ShapeDtypeStruct((_TOKENS, _V), jnp.float32),
            jax.ShapeDtypeStruct((_TOKENS, 1), jnp.float32),
            jax.ShapeDtypeStruct((_TOKENS, 1), jnp.int32),
        ],
        scratch_shapes=[
            pltpu.VMEM((_BM, 1), jnp.float32),
            pltpu.VMEM((_BM, 1), jnp.float32),
            pltpu.VMEM((_BM, 1), jnp.float32),
            pltpu.VMEM((_BM, 1), jnp.float32),
            pltpu.VMEM((_BM, 1), jnp.int32),
        ],
        compiler_params=pltpu.CompilerParams(
            dimension_semantics=("parallel", "arbitrary")),
    )(x, W, b2d)

    sample = pl.pallas_call(
        _onehot_kernel,
        grid=(_NT2,),
        in_specs=[pl.BlockSpec((_TOKENS, 1), lambda j: (0, 0))],
        out_specs=pl.BlockSpec((_TOKENS, _BN2), lambda j: (0, j)),
        out_shape=jax.ShapeDtypeStruct((_TOKENS, _V), jnp.float32),
        compiler_params=pltpu.CompilerParams(
            dimension_semantics=("arbitrary",)),
    )(idx)

    return (sample, logits, ent.reshape(_TOKENS))


# speedup vs baseline: 1.0314x; 1.0314x over previous
import jax
import jax.numpy as jnp
from jax import lax
from jax.experimental import pallas as pl
from jax.experimental.pallas import tpu as pltpu

_TOKENS = 1024
_D = 2048
_V = 32768
_BN = 1024
_NT = _V // _BN
_NM = 1
_BM = _TOKENS // _NM
_BN2 = 2048
_NT2 = _V // _BN2

_KS0 = 0
_KS1 = 42
_KS2 = _KS0 ^ _KS1 ^ 0x1BD11BDA


def _threefry_bits(flat):
    ks = (jnp.uint32(_KS0), jnp.uint32(_KS1), jnp.uint32(_KS2))
    rots = ((13, 15, 26, 6), (17, 29, 16, 24))
    x0 = jnp.full_like(flat, ks[0])
    x1 = flat + ks[1]
    for i in range(5):
        for r in rots[i % 2]:
            x0 = x0 + x1
            x1 = lax.shift_left(x1, jnp.uint32(r)) | lax.shift_right_logical(
                x1, jnp.uint32(32 - r))
            x1 = x0 ^ x1
        x0 = x0 + ks[(i + 1) % 3]
        x1 = x1 + ks[(i + 2) % 3] + jnp.uint32(i + 1)
    return x0 ^ x1


def _gumbel(fiota, j):
    flat = fiota + jnp.asarray(j * _BN, jnp.uint32)
    bits = _threefry_bits(flat)
    fb = lax.shift_right_logical(bits, jnp.uint32(9)) | jnp.uint32(0x3F800000)
    u = lax.bitcast_convert_type(fb, jnp.float32) - jnp.float32(1.0)
    mn = jnp.float32(1e-10)
    u = jnp.maximum(mn, u * (jnp.float32(1.0) - mn) + mn)
    return -jnp.log(-jnp.log(u))


def _main_kernel(x_ref, w_ref, b_ref, logits_ref, ent_ref, idx_ref,
                 m_ref, s_ref, t_ref, bv_ref, bi_ref, fiota_ref):
    j = pl.program_id(1)

    @pl.when(j == 0)
    def _():
        m_ref[...] = jnp.full_like(m_ref, -jnp.inf)
        s_ref[...] = jnp.zeros_like(s_ref)
        t_ref[...] = jnp.zeros_like(t_ref)
        bv_ref[...] = jnp.full_like(bv_ref, -jnp.inf)
        bi_ref[...] = jnp.zeros_like(bi_ref)
        r = lax.broadcasted_iota(jnp.uint32, (_BM, _BN), 0)
        c = lax.broadcasted_iota(jnp.uint32, (_BM, _BN), 1)
        fiota_ref[...] = r * jnp.uint32(_V) + c

    l = jnp.dot(x_ref[...], w_ref[...],
                preferred_element_type=jnp.float32) + b_ref[...]
    logits_ref[...] = l

    z = l + _gumbel(fiota_ref[...], j)
    cg = lax.broadcasted_iota(jnp.int32, (_BM, _BN), 1) + j * _BN
    upd = z > bv_ref[...]
    bv_ref[...] = jnp.where(upd, z, bv_ref[...])
    bi_ref[...] = jnp.where(upd, cg, bi_ref[...])

    m_old = m_ref[...]
    m_new = jnp.maximum(m_old, jnp.max(l, axis=1, keepdims=True))
    e = jnp.exp(l - m_new)
    alpha = jnp.exp(m_old - m_new)
    s_ref[...] = s_ref[...] * alpha + jnp.sum(e, axis=1, keepdims=True)
    t_ref[...] = t_ref[...] * alpha + jnp.sum(l * e, axis=1, keepdims=True)
    m_ref[...] = m_new

    @pl.when(j == _NT - 1)
    def _():
        s = s_ref[...]
        ent_ref[...] = m_ref[...] + jnp.log(s) - t_ref[...] / s
        bv = bv_ref[...]
        gm = jnp.max(bv, axis=1, keepdims=True)
        idx_ref[...] = jnp.min(
            jnp.where(bv >= gm, bi_ref[...], jnp.int32(2**31 - 1)),
            axis=1, keepdims=True)


def _onehot_kernel(idx_ref, out_ref):
    j = pl.program_id(0)
    cg = lax.broadcasted_iota(jnp.int32, (_TOKENS, _BN2), 1) + j * _BN2
    out_ref[...] = (cg == idx_ref[...]).astype(jnp.float32)


def kernel(x, W, b):
    b2d = b.reshape(1, _V)
    logits, ent, idx = pl.pallas_call(
        _main_kernel,
        grid=(_NM, _NT),
        in_specs=[
            pl.BlockSpec((_BM, _D), lambda i, j: (i, 0)),
            pl.BlockSpec((_D, _BN), lambda i, j: (0, j)),
            pl.BlockSpec((1, _BN), lambda i, j: (0, j)),
        ],
        out_specs=[
            pl.BlockSpec((_BM, _BN), lambda i, j: (i, j)),
            pl.BlockSpec((_BM, 1), lambda i, j: (i, 0)),
            pl.BlockSpec((_BM, 1), lambda i, j: (i, 0)),
        ],
        out_shape=[
            jax.ShapeDtypeStruct((_TOKENS, _V), jnp.float32),
            jax.ShapeDtypeStruct((_TOKENS, 1), jnp.float32),
            jax.ShapeDtypeStruct((_TOKENS, 1), jnp.int32),
        ],
        scratch_shapes=[
            pltpu.VMEM((_BM, 1), jnp.float32),
            pltpu.VMEM((_BM, 1), jnp.float32),
            pltpu.VMEM((_BM, 1), jnp.float32),
            pltpu.VMEM((_BM, _BN), jnp.float32),
            pltpu.VMEM((_BM, _BN), jnp.int32),
            pltpu.VMEM((_BM, _BN), jnp.uint32),
        ],
        compiler_params=pltpu.CompilerParams(
            dimension_semantics=("parallel", "arbitrary")),
    )(x, W, b2d)

    sample = pl.pallas_call(
        _onehot_kernel,
        grid=(_NT2,),
        in_specs=[pl.BlockSpec((_TOKENS, 1), lambda j: (0, 0))],
        out_specs=pl.BlockSpec((_TOKENS, _BN2), lambda j: (0, j)),
        out_shape=jax.ShapeDtypeStruct((_TOKENS, _V), jnp.float32),
        compiler_params=pltpu.CompilerParams(
            dimension_semantics=("arbitrary",)),
    )(idx)

    return (sample, logits, ent.reshape(_TOKENS))


# speedup vs baseline: 1.0514x; 1.0194x over previous
import jax
import jax.numpy as jnp
from jax import lax
from jax.experimental import pallas as pl
from jax.experimental.pallas import tpu as pltpu

_TOKENS = 1024
_D = 2048
_V = 32768
_BN = 1024
_NT = _V // _BN
_BN2 = 2048
_NT2 = _V // _BN2

_KS0 = 0
_KS1 = 42
_KS2 = _KS0 ^ _KS1 ^ 0x1BD11BDA


def _threefry_bits(flat):
    ks = (_KS0, _KS1, _KS2)
    rots = ((13, 15, 26, 6), (17, 29, 16, 24))
    x1 = flat + jnp.uint32(ks[1])
    x0 = None
    for i in range(5):
        for r in rots[i % 2]:
            x0 = x1 if x0 is None else x0 + x1
            x1 = lax.shift_left(x1, jnp.uint32(r)) | lax.shift_right_logical(
                x1, jnp.uint32(32 - r))
            x1 = x0 ^ x1
        x0 = x0 + jnp.uint32(ks[(i + 1) % 3])
        x1 = x1 + jnp.uint32((ks[(i + 2) % 3] + i + 1) & 0xFFFFFFFF)
    return x0 ^ x1


def _gumbel(j):
    r = lax.broadcasted_iota(jnp.uint32, (_TOKENS, _BN), 0)
    c = lax.broadcasted_iota(jnp.uint32, (_TOKENS, _BN), 1)
    cglob = c + jnp.asarray(j * _BN, jnp.uint32)
    flat = r * jnp.uint32(_V) + cglob
    bits = _threefry_bits(flat)
    fb = lax.shift_right_logical(bits, jnp.uint32(9)) | jnp.uint32(0x3F800000)
    f = lax.bitcast_convert_type(fb, jnp.float32) - jnp.float32(1.0)
    mn = jnp.float32(1e-10)
    u = jnp.maximum(mn, f + mn)
    return -jnp.log(-jnp.log(u)), lax.bitcast_convert_type(cglob, jnp.int32)


def _main_kernel(x_ref, w_ref, b_ref, logits_ref, ent_ref, idx_ref,
                 m_ref, s_ref, t_ref, bv_ref, bi_ref):
    j = pl.program_id(0)
    l = jnp.dot(x_ref[...], w_ref[...],
                preferred_element_type=jnp.float32) + b_ref[...]
    logits_ref[...] = l

    g, cg = _gumbel(j)
    z = l + g
    ztm = jnp.max(z, axis=1, keepdims=True)
    zarg = jnp.min(jnp.where(z >= ztm, cg, jnp.int32(2**31 - 1)),
                   axis=1, keepdims=True)

    @pl.when(j == 0)
    def _():
        m_ref[...] = jnp.full_like(m_ref, -jnp.inf)
        s_ref[...] = jnp.zeros_like(s_ref)
        t_ref[...] = jnp.zeros_like(t_ref)
        bv_ref[...] = jnp.full_like(bv_ref, -jnp.inf)
        bi_ref[...] = jnp.zeros_like(bi_ref)

    m_old = m_ref[...]
    m_new = jnp.maximum(m_old, jnp.max(l, axis=1, keepdims=True))
    e = jnp.exp(l - m_new)
    alpha = jnp.exp(m_old - m_new)
    ones = jnp.ones((_BN, 8), jnp.float32)
    se = jnp.dot(e, ones, preferred_element_type=jnp.float32)[:, 0:1]
    te = jnp.dot(l * e, ones, preferred_element_type=jnp.float32)[:, 0:1]
    s_ref[...] = s_ref[...] * alpha + se
    t_ref[...] = t_ref[...] * alpha + te
    m_ref[...] = m_new

    better = ztm > bv_ref[...]
    bv_ref[...] = jnp.where(better, ztm, bv_ref[...])
    bi_ref[...] = jnp.where(better, zarg, bi_ref[...])

    @pl.when(j == _NT - 1)
    def _():
        s = s_ref[...]
        ent_ref[...] = m_ref[...] + jnp.log(s) - t_ref[...] / s
        idx_ref[...] = bi_ref[...]


def _onehot_kernel(idx_ref, out_ref):
    j = pl.program_id(0)
    cg = lax.broadcasted_iota(jnp.int32, (_TOKENS, _BN2), 1) + j * _BN2
    out_ref[...] = (cg == idx_ref[...]).astype(jnp.float32)


def kernel(x, W, b):
    b2d = b.reshape(1, _V)
    logits, ent, idx = pl.pallas_call(
        _main_kernel,
        grid=(_NT,),
        in_specs=[
            pl.BlockSpec((_TOKENS, _D), lambda j: (0, 0)),
            pl.BlockSpec((_D, _BN), lambda j: (0, j)),
            pl.BlockSpec((1, _BN), lambda j: (0, j)),
        ],
        out_specs=[
            pl.BlockSpec((_TOKENS, _BN), lambda j: (0, j)),
            pl.BlockSpec((_TOKENS, 1), lambda j: (0, 0)),
            pl.BlockSpec((_TOKENS, 1), lambda j: (0, 0)),
        ],
        out_shape=[
            jax.ShapeDtypeStruct((_TOKENS, _V), jnp.float32),
            jax.ShapeDtypeStruct((_TOKENS, 1), jnp.float32),
            jax.ShapeDtypeStruct((_TOKENS, 1), jnp.int32),
        ],
        scratch_shapes=[
            pltpu.VMEM((_TOKENS, 1), jnp.float32),
            pltpu.VMEM((_TOKENS, 1), jnp.float32),
            pltpu.VMEM((_TOKENS, 1), jnp.float32),
            pltpu.VMEM((_TOKENS, 1), jnp.float32),
            pltpu.VMEM((_TOKENS, 1), jnp.int32),
        ],
        compiler_params=pltpu.CompilerParams(
            dimension_semantics=("arbitrary",)),
    )(x, W, b2d)

    sample = pl.pallas_call(
        _onehot_kernel,
        grid=(_NT2,),
        in_specs=[pl.BlockSpec((_TOKENS, 1), lambda j: (0, 0))],
        out_specs=pl.BlockSpec((_TOKENS, _BN2), lambda j: (0, j)),
        out_shape=jax.ShapeDtypeStruct((_TOKENS, _V), jnp.float32),
        compiler_params=pltpu.CompilerParams(
            dimension_semantics=("arbitrary",)),
    )(idx)

    return (sample, logits, ent.reshape(_TOKENS))
